# baseline (speedup 1.0000x reference)
import functools

import jax
import jax.numpy as jnp
from jax import lax
from jax.experimental import pallas as pl
from jax.experimental.pallas import tpu as pltpu
from jax.experimental.pallas import tpu_sc as plsc

NUM_K = 1024
DIM = 64
N_B = 16
N_HW = 32 * 32
N_TOK = N_B * N_HW

SC_NC = 2
SC_NS = 16
SC_NW = SC_NC * SC_NS
SC_CHUNK = N_TOK // SC_NW


def _vq_argmin_kernel(z_ref, e_ref, esq_ref, idx_ref, loss_ref, perp_ref,
                      used_ref, counts_acc, sse_acc):
    i = pl.program_id(0)
    zb = z_ref[0]
    emb = e_ref[...]
    esq = esq_ref[...]
    mm = jax.lax.dot_general(emb, zb, (((1,), (0,)), ((), ())),
                             preferred_element_type=jnp.float32)
    zsq = jnp.sum(zb * zb, axis=0, keepdims=True)
    dist = (zsq + esq) - 2.0 * mm
    minv = jnp.min(dist, axis=0, keepdims=True)
    iota_k = jax.lax.broadcasted_iota(jnp.int32, dist.shape, 0)
    idx = jnp.min(jnp.where(dist <= minv, iota_k, NUM_K),
                  axis=0, keepdims=True)
    idx_ref[...] = idx[None]

    onehot = (iota_k == idx).astype(jnp.float32)
    tile_counts = jnp.sum(onehot, axis=1, keepdims=True)
    tile_sse = jnp.sum(minv)

    @pl.when(i == 0)
    def _init():
        counts_acc[...] = tile_counts
        sse_acc[0, 0] = tile_sse

    @pl.when(i > 0)
    def _accum():
        counts_acc[...] += tile_counts
        sse_acc[0, 0] += tile_sse

    @pl.when(i == N_B - 1)
    def _finalize():
        counts = counts_acc[...]
        p = counts * (1.0 / N_TOK)
        perp = jnp.exp(-jnp.sum(p * jnp.log(p + 1e-10)))
        perp_ref[...] = jnp.reshape(perp, (1, 1))
        used_ref[...] = (counts > 0).astype(jnp.float32)
        loss_ref[...] = jnp.reshape(sse_acc[0, 0] * (1.0 / (N_TOK * DIM)), (1, 1))


def _sc_gather(idx_hbm, emb_hbm, q_hbm, idx_v, rows_v, sem):
    wid = lax.axis_index("s") * SC_NC + lax.axis_index("c")
    base = wid * SC_CHUNK
    pltpu.sync_copy(idx_hbm.at[pl.ds(base, SC_CHUNK)], idx_v)
    copies = []
    for j in range(SC_CHUNK // 128):
        copies.append(pltpu.async_copy(
            emb_hbm.at[idx_v.at[pl.ds(j * 128, 128)]],
            rows_v.at[pl.ds(j * 128, 128)], sem))
    for c in copies:
        c.wait()
    pltpu.sync_copy(rows_v, q_hbm.at[pl.ds(base, SC_CHUNK)])


def kernel(z, embedding):
    B, D, H, W = z.shape
    z3 = z.reshape(B, D, H * W)
    esq_col = jnp.sum(embedding ** 2, axis=1, keepdims=True)

    idx3, loss, perp, used = pl.pallas_call(
        _vq_argmin_kernel,
        grid=(N_B,),
        in_specs=[
            pl.BlockSpec((1, DIM, N_HW), lambda i: (i, 0, 0)),
            pl.BlockSpec((NUM_K, DIM), lambda i: (0, 0)),
            pl.BlockSpec((NUM_K, 1), lambda i: (0, 0)),
        ],
        out_specs=[
            pl.BlockSpec((1, 1, N_HW), lambda i: (i, 0, 0)),
            pl.BlockSpec((1, 1), lambda i: (0, 0)),
            pl.BlockSpec((1, 1), lambda i: (0, 0)),
            pl.BlockSpec((NUM_K, 1), lambda i: (0, 0)),
        ],
        out_shape=[
            jax.ShapeDtypeStruct((N_B, 1, N_HW), jnp.int32),
            jax.ShapeDtypeStruct((1, 1), jnp.float32),
            jax.ShapeDtypeStruct((1, 1), jnp.float32),
            jax.ShapeDtypeStruct((NUM_K, 1), jnp.float32),
        ],
        scratch_shapes=[
            pltpu.VMEM((NUM_K, 1), jnp.float32),
            pltpu.SMEM((1, 1), jnp.float32),
        ],
    )(z3, embedding, esq_col)

    idx_flat = idx3.reshape(N_TOK)
    emb_pad = jnp.pad(embedding, ((0, 0), (0, 128 - DIM)))

    sc_fn = functools.partial(
        pl.kernel,
        mesh=plsc.VectorSubcoreMesh(core_axis_name="c", subcore_axis_name="s"),
        out_type=jax.ShapeDtypeStruct((N_TOK, 128), jnp.float32),
        scratch_types=[
            pltpu.VMEM((SC_CHUNK,), jnp.int32),
            pltpu.VMEM((SC_CHUNK, 128), jnp.float32),
            pltpu.SemaphoreType.DMA,
        ],
    )(_sc_gather)
    q_rows = sc_fn(idx_flat, emb_pad)

    z_q = jnp.transpose(q_rows[:, :DIM].reshape(B, H, W, D), (0, 3, 1, 2))
    indices = idx3.reshape(B, H, W)
    return (z_q, loss.reshape(()), indices, perp.reshape(()), used.reshape(NUM_K))


# speedup vs baseline: 1.4263x; 1.4263x over previous
import jax
import jax.numpy as jnp
from jax.experimental import pallas as pl
from jax.experimental.pallas import tpu as pltpu

NUM_K = 1024
DIM = 64
N_B = 16
N_HW = 32 * 32
N_TOK = N_B * N_HW


def _vq_tc_kernel(z_ref, e_ref, esq_ref, idx_ref, q_ref, loss_ref, perp_ref,
                  used_ref, counts_acc, sse_acc):
    i = pl.program_id(0)
    zb = z_ref[0]
    emb = e_ref[...]
    esq = esq_ref[...]
    mm = jax.lax.dot_general(emb, zb, (((1,), (0,)), ((), ())),
                             preferred_element_type=jnp.float32)
    zsq = jnp.sum(zb * zb, axis=0, keepdims=True)
    dist = (zsq + esq) - 2.0 * mm
    minv = jnp.min(dist, axis=0, keepdims=True)
    iota_k = jax.lax.broadcasted_iota(jnp.int32, dist.shape, 0)
    idx = jnp.min(jnp.where(dist <= minv, iota_k, NUM_K),
                  axis=0, keepdims=True)
    idx_ref[...] = idx[None]

    onehot = (iota_k == idx).astype(jnp.float32)
    q_ref[...] = jax.lax.dot_general(emb, onehot, (((0,), (0,)), ((), ())),
                                     preferred_element_type=jnp.float32)[None]

    tile_counts = jnp.sum(onehot, axis=1, keepdims=True)
    tile_sse = jnp.sum(minv)

    @pl.when(i == 0)
    def _init():
        counts_acc[...] = tile_counts
        sse_acc[0, 0] = tile_sse

    @pl.when(i > 0)
    def _accum():
        counts_acc[...] += tile_counts
        sse_acc[0, 0] += tile_sse

    @pl.when(i == N_B - 1)
    def _finalize():
        counts = counts_acc[...]
        p = counts * (1.0 / N_TOK)
        perp = jnp.exp(-jnp.sum(p * jnp.log(p + 1e-10)))
        perp_ref[...] = jnp.reshape(perp, (1, 1))
        used_ref[...] = (counts > 0).astype(jnp.float32)
        loss_ref[...] = jnp.reshape(sse_acc[0, 0] * (1.0 / (N_TOK * DIM)), (1, 1))


def kernel(z, embedding):
    B, D, H, W = z.shape
    z3 = z.reshape(B, D, H * W)
    esq_col = jnp.sum(embedding ** 2, axis=1, keepdims=True)

    idx3, q3, loss, perp, used = pl.pallas_call(
        _vq_tc_kernel,
        grid=(N_B,),
        in_specs=[
            pl.BlockSpec((1, DIM, N_HW), lambda i: (i, 0, 0)),
            pl.BlockSpec((NUM_K, DIM), lambda i: (0, 0)),
            pl.BlockSpec((NUM_K, 1), lambda i: (0, 0)),
        ],
        out_specs=[
            pl.BlockSpec((1, 1, N_HW), lambda i: (i, 0, 0)),
            pl.BlockSpec((1, DIM, N_HW), lambda i: (i, 0, 0)),
            pl.BlockSpec((1, 1), lambda i: (0, 0)),
            pl.BlockSpec((1, 1), lambda i: (0, 0)),
            pl.BlockSpec((NUM_K, 1), lambda i: (0, 0)),
        ],
        out_shape=[
            jax.ShapeDtypeStruct((N_B, 1, N_HW), jnp.int32),
            jax.ShapeDtypeStruct((N_B, DIM, N_HW), jnp.float32),
            jax.ShapeDtypeStruct((1, 1), jnp.float32),
            jax.ShapeDtypeStruct((1, 1), jnp.float32),
            jax.ShapeDtypeStruct((NUM_K, 1), jnp.float32),
        ],
        scratch_shapes=[
            pltpu.VMEM((NUM_K, 1), jnp.float32),
            pltpu.SMEM((1, 1), jnp.float32),
        ],
    )(z3, embedding, esq_col)

    z_q = q3.reshape(B, D, H, W)
    indices = idx3.reshape(B, H, W)
    return (z_q, loss.reshape(()), indices, perp.reshape(()), used.reshape(NUM_K))
